# baseline (speedup 1.0000x reference)
import functools

import jax
import jax.numpy as jnp
from jax import lax
from jax.experimental import pallas as pl
from jax.experimental.pallas import tpu as pltpu
from jax.experimental.pallas import tpu_sc as plsc

N = 1_000_000
LANES = 16
NW = 32

BLK_TC = 32_768
NBLK_TC = 24
M = BLK_TC * NBLK_TC
S = N - M
CHUNK = (S - 64) // NW
TAIL = 64
TAIL_BASE = N - TAIL


def _hc_vec(a, u):
    g = jnp.exp(-a) * (1.0 - u) / u
    i = lax.bitcast_convert_type(g, jnp.int32)
    i = jnp.int32(0x5F3759DF) - lax.shift_right_logical(i, 1)
    y = lax.bitcast_convert_type(i, jnp.float32)
    y = y * (1.5 - 0.5 * g * y * y)
    y = y * (1.5 - 0.5 * g * y * y)
    z = g * g * y
    s = (1.1 - 0.1 * z) / (1.0 + z)
    return jnp.clip(s, 0.0, 1.0)


def _tc_body(a_ref, u_ref, o_ref):
    o_ref[...] = _hc_vec(a_ref[...], u_ref[...])


_tc_kernel = pl.pallas_call(
    _tc_body,
    grid=(NBLK_TC,),
    in_specs=[
        pl.BlockSpec((BLK_TC,), lambda i: (i,)),
        pl.BlockSpec((BLK_TC,), lambda i: (i,)),
    ],
    out_specs=pl.BlockSpec((BLK_TC,), lambda i: (i,)),
    out_shape=jax.ShapeDtypeStruct((M,), jnp.float32),
)


_mesh = plsc.VectorSubcoreMesh(core_axis_name="c", subcore_axis_name="s")


@functools.partial(
    pl.kernel,
    mesh=_mesh,
    out_type=jax.ShapeDtypeStruct((S,), jnp.float32),
    scratch_types=[
        pltpu.VMEM((CHUNK,), jnp.float32),
        pltpu.VMEM((CHUNK,), jnp.float32),
        pltpu.VMEM((CHUNK,), jnp.float32),
        pltpu.VMEM((TAIL,), jnp.float32),
        pltpu.VMEM((TAIL,), jnp.float32),
        pltpu.VMEM((TAIL,), jnp.float32),
    ],
)
def _sc_kernel(a_hbm, u_hbm, o_hbm, a_v, u_v, o_v, at_v, ut_v, ot_v):
    wid = lax.axis_index("s") * 2 + lax.axis_index("c")
    base = M + wid * CHUNK
    obase = wid * CHUNK
    pltpu.sync_copy(a_hbm.at[pl.ds(base, CHUNK)], a_v)
    pltpu.sync_copy(u_hbm.at[pl.ds(base, CHUNK)], u_v)

    @plsc.parallel_loop(0, CHUNK, step=LANES, unroll=8)
    def _compute(i):
        sl = pl.ds(i, LANES)
        o_v[sl] = _hc_vec(a_v[sl], u_v[sl])

    pltpu.sync_copy(o_v, o_hbm.at[pl.ds(obase, CHUNK)])

    @pl.when(wid == 0)
    def _tail():
        pltpu.sync_copy(a_hbm.at[pl.ds(TAIL_BASE, TAIL)], at_v)
        pltpu.sync_copy(u_hbm.at[pl.ds(TAIL_BASE, TAIL)], ut_v)
        for j in range(TAIL // LANES):
            sl = pl.ds(j * LANES, LANES)
            ot_v[sl] = _hc_vec(at_v[sl], ut_v[sl])
        pltpu.sync_copy(ot_v, o_hbm.at[pl.ds(S - TAIL, TAIL)])


def kernel(log_alpha, u, current_iter):
    sc_out = _sc_kernel(log_alpha, u)
    tc_out = _tc_kernel(log_alpha, u)
    return jnp.concatenate([tc_out, sc_out])


# speedup vs baseline: 1.2127x; 1.2127x over previous
import functools

import jax
import jax.numpy as jnp
from jax import lax
from jax.experimental import pallas as pl
from jax.experimental.pallas import tpu as pltpu
from jax.experimental.pallas import tpu_sc as plsc

N = 1_000_000
LANES = 16
NW = 32

BLK_TC = 32_768
NBLK_TC = 24
M = BLK_TC * NBLK_TC
S = N - M
CHUNK = (S - 64) // NW
TAIL = 64
TAIL_BASE = N - TAIL


def _hc_vec(a, u):
    g = jnp.exp(-a) * (1.0 - u) / u
    i = lax.bitcast_convert_type(g, jnp.int32)
    i = jnp.int32(0x5F3759DF) - lax.shift_right_logical(i, 1)
    y = lax.bitcast_convert_type(i, jnp.float32)
    y = y * (1.5 - 0.5 * g * y * y)
    y = y * (1.5 - 0.5 * g * y * y)
    z = g * g * y
    s = (1.1 - 0.1 * z) / (1.0 + z)
    return jnp.clip(s, 0.0, 1.0)


def _tc_body(a_ref, u_ref, o_ref):
    a = a_ref[...]
    u = u_ref[...]
    g = jnp.exp(-a) * (1.0 - u) / u
    z = g * g * lax.rsqrt(g)
    s = (1.1 - 0.1 * z) / (1.0 + z)
    o_ref[...] = jnp.clip(s, 0.0, 1.0)


_tc_kernel = pl.pallas_call(
    _tc_body,
    grid=(NBLK_TC,),
    in_specs=[
        pl.BlockSpec((BLK_TC,), lambda i: (i,)),
        pl.BlockSpec((BLK_TC,), lambda i: (i,)),
    ],
    out_specs=pl.BlockSpec((BLK_TC,), lambda i: (i,)),
    out_shape=jax.ShapeDtypeStruct((N,), jnp.float32),
)


_mesh = plsc.VectorSubcoreMesh(core_axis_name="c", subcore_axis_name="s")


@functools.partial(
    pl.kernel,
    mesh=_mesh,
    out_type=jax.ShapeDtypeStruct((S,), jnp.float32),
    scratch_types=[
        pltpu.VMEM((CHUNK,), jnp.float32),
        pltpu.VMEM((CHUNK,), jnp.float32),
        pltpu.VMEM((CHUNK,), jnp.float32),
        pltpu.VMEM((TAIL,), jnp.float32),
        pltpu.VMEM((TAIL,), jnp.float32),
        pltpu.VMEM((TAIL,), jnp.float32),
    ],
)
def _sc_kernel(a_hbm, u_hbm, o_hbm, a_v, u_v, o_v, at_v, ut_v, ot_v):
    wid = lax.axis_index("s") * 2 + lax.axis_index("c")
    base = M + wid * CHUNK
    obase = wid * CHUNK
    pltpu.sync_copy(a_hbm.at[pl.ds(base, CHUNK)], a_v)
    pltpu.sync_copy(u_hbm.at[pl.ds(base, CHUNK)], u_v)

    @plsc.parallel_loop(0, CHUNK, step=LANES, unroll=8)
    def _compute(i):
        sl = pl.ds(i, LANES)
        o_v[sl] = _hc_vec(a_v[sl], u_v[sl])

    pltpu.sync_copy(o_v, o_hbm.at[pl.ds(obase, CHUNK)])

    @pl.when(wid == 0)
    def _tail():
        pltpu.sync_copy(a_hbm.at[pl.ds(TAIL_BASE, TAIL)], at_v)
        pltpu.sync_copy(u_hbm.at[pl.ds(TAIL_BASE, TAIL)], ut_v)
        for j in range(TAIL // LANES):
            sl = pl.ds(j * LANES, LANES)
            ot_v[sl] = _hc_vec(at_v[sl], ut_v[sl])
        pltpu.sync_copy(ot_v, o_hbm.at[pl.ds(S - TAIL, TAIL)])


def kernel(log_alpha, u, current_iter):
    sc_out = _sc_kernel(log_alpha, u)
    tc_out = _tc_kernel(log_alpha, u)
    return lax.dynamic_update_slice(tc_out, sc_out, (M,))


# speedup vs baseline: 1.4273x; 1.1770x over previous
import functools

import jax
import jax.numpy as jnp
from jax import lax
from jax.experimental import pallas as pl
from jax.experimental.pallas import tpu as pltpu
from jax.experimental.pallas import tpu_sc as plsc

N = 1_000_000
LANES = 16
NW = 32

BLK_TC = 98_304
NBLK_TC = 8
M = BLK_TC * NBLK_TC
S = N - M
CHUNK = (S - 64) // NW
TAIL = 64
TAIL_BASE = N - TAIL


def _hc_vec(a, u):
    g = jnp.exp(-a) * (1.0 - u) / u
    i = lax.bitcast_convert_type(g, jnp.int32)
    i = jnp.int32(0x5F3759DF) - lax.shift_right_logical(i, 1)
    y = lax.bitcast_convert_type(i, jnp.float32)
    y = y * (1.5 - 0.5 * g * y * y)
    y = y * (1.5 - 0.5 * g * y * y)
    z = g * g * y
    s = (1.1 - 0.1 * z) / (1.0 + z)
    return jnp.clip(s, 0.0, 1.0)


def _tc_body(a_ref, u_ref, o_ref):
    a = a_ref[...]
    u = u_ref[...]
    g = jnp.exp(-a) * (1.0 - u) / u
    z = g * g * lax.rsqrt(g)
    s = (1.1 - 0.1 * z) / (1.0 + z)
    o_ref[...] = jnp.clip(s, 0.0, 1.0)


_tc_kernel = pl.pallas_call(
    _tc_body,
    grid=(NBLK_TC,),
    in_specs=[
        pl.BlockSpec((BLK_TC,), lambda i: (i,)),
        pl.BlockSpec((BLK_TC,), lambda i: (i,)),
    ],
    out_specs=pl.BlockSpec((BLK_TC,), lambda i: (i,)),
    out_shape=jax.ShapeDtypeStruct((N,), jnp.float32),
)


_mesh = plsc.VectorSubcoreMesh(core_axis_name="c", subcore_axis_name="s")


@functools.partial(
    pl.kernel,
    mesh=_mesh,
    out_type=jax.ShapeDtypeStruct((S,), jnp.float32),
    scratch_types=[
        pltpu.VMEM((CHUNK,), jnp.float32),
        pltpu.VMEM((CHUNK,), jnp.float32),
        pltpu.VMEM((CHUNK,), jnp.float32),
        pltpu.VMEM((TAIL,), jnp.float32),
        pltpu.VMEM((TAIL,), jnp.float32),
        pltpu.VMEM((TAIL,), jnp.float32),
    ],
)
def _sc_kernel(a_hbm, u_hbm, o_hbm, a_v, u_v, o_v, at_v, ut_v, ot_v):
    wid = lax.axis_index("s") * 2 + lax.axis_index("c")
    base = M + wid * CHUNK
    obase = wid * CHUNK
    pltpu.sync_copy(a_hbm.at[pl.ds(base, CHUNK)], a_v)
    pltpu.sync_copy(u_hbm.at[pl.ds(base, CHUNK)], u_v)

    @plsc.parallel_loop(0, CHUNK, step=LANES, unroll=8)
    def _compute(i):
        sl = pl.ds(i, LANES)
        o_v[sl] = _hc_vec(a_v[sl], u_v[sl])

    pltpu.sync_copy(o_v, o_hbm.at[pl.ds(obase, CHUNK)])

    @pl.when(wid == 0)
    def _tail():
        pltpu.sync_copy(a_hbm.at[pl.ds(TAIL_BASE, TAIL)], at_v)
        pltpu.sync_copy(u_hbm.at[pl.ds(TAIL_BASE, TAIL)], ut_v)
        for j in range(TAIL // LANES):
            sl = pl.ds(j * LANES, LANES)
            ot_v[sl] = _hc_vec(at_v[sl], ut_v[sl])
        pltpu.sync_copy(ot_v, o_hbm.at[pl.ds(S - TAIL, TAIL)])


def kernel(log_alpha, u, current_iter):
    sc_out = _sc_kernel(log_alpha, u)
    tc_out = _tc_kernel(log_alpha, u)
    return lax.dynamic_update_slice(tc_out, sc_out, (M,))
